# baseline (speedup 1.0000x reference)
import jax, jax.numpy as jnp
from jax.experimental import pallas as pl

PATCH = 16
EMB = 16
CF = 4
K = 8192
H = 64
RH = 32
NRES = 2
BETA = 0.25
CODE_DIM = EMB * (PATCH // CF)

RBLK = 512
EBLK = 2048
N_TOTAL = 28672


def _bf(v):
    return v.astype(jnp.bfloat16)


def _enc_kernel(x_ref, w1_ref, b1_ref, w2_ref, b2_ref, w3_ref, b3_ref,
                wa0_ref, ba0_ref, wb0_ref, bb0_ref, wa1_ref, ba1_ref,
                wb1_ref, bb1_ref, wp_ref, bp_ref, z_ref):
    xb = _bf(x_ref[...])
    n = xb.shape[0]
    zero1 = jnp.zeros((n, 1), jnp.bfloat16)

    def dot(a, w):
        return jax.lax.dot_general(a, w, dimension_numbers=(((1,), (0,)), ((), ())),
                                   preferred_element_type=jnp.float32)

    w1 = w1_ref[...]
    b1 = b1_ref[...]
    h1 = []
    for lo in range(8):
        if lo == 0:
            win = jnp.concatenate([zero1, xb[:, 0:3]], axis=1)
        elif lo == 7:
            win = jnp.concatenate([xb[:, 13:16], zero1], axis=1)
        else:
            win = xb[:, 2 * lo - 1:2 * lo + 3]
        h1.append(_bf(jax.nn.relu(dot(win, w1) + b1)))

    w2 = w2_ref[...]
    b2 = b2_ref[...]
    zero32 = jnp.zeros((n, 32), jnp.bfloat16)
    h1p = [zero32] + h1 + [zero32]
    h2 = []
    for lo in range(4):
        win = jnp.concatenate(h1p[2 * lo:2 * lo + 4], axis=1)
        h2.append(_bf(jax.nn.relu(dot(win, w2) + b2)))

    w3 = w3_ref[...]
    b3 = b3_ref[...]
    zero64 = jnp.zeros((n, 64), jnp.bfloat16)
    h2p = [zero64] + h2 + [zero64]
    h = [dot(jnp.concatenate(h2p[lo:lo + 3], axis=1), w3) + b3 for lo in range(4)]

    for wa_ref, ba_ref, wb_ref, bb_ref in ((wa0_ref, ba0_ref, wb0_ref, bb0_ref),
                                           (wa1_ref, ba1_ref, wb1_ref, bb1_ref)):
        wa = wa_ref[...]
        ba = ba_ref[...]
        wb = wb_ref[...]
        bb = bb_ref[...]
        rin = [_bf(jax.nn.relu(t)) for t in h]
        rinp = [zero64] + rin + [zero64]
        s = [_bf(jax.nn.relu(dot(jnp.concatenate(rinp[lo:lo + 3], axis=1), wa) + ba))
             for lo in range(4)]
        r = [dot(t, wb) + bb for t in s]
        h = [u + v for u, v in zip(h, r)]

    bpf = bp_ref[...]
    wp = wp_ref[...]
    r = [_bf(jax.nn.relu(t)) for t in h]
    acc = dot(r[0], wp[0])
    for pos in range(1, 4):
        acc = acc + dot(r[pos], wp[pos])
    z_ref[...] = acc + bpf


def _encode(xpatch, params):
    p = params

    def tapmat(w):
        return _bf(w.transpose(2, 1, 0).reshape(-1, w.shape[0]))

    wpt = tapmat(p['wp'])
    wps = jnp.zeros((4, H, CODE_DIM), jnp.bfloat16)
    wps = wps.at[jnp.arange(4)[:, None], :, jnp.arange(EMB)[None, :] * CF
                 + jnp.arange(4)[:, None]].set(wpt.T[None, :, :].repeat(4, 0))
    bpf = jnp.zeros((1, CODE_DIM), jnp.float32)
    bpf = bpf.at[0, jnp.arange(EMB)[:, None] * CF + jnp.arange(4)[None, :]].set(
        p['bp'][:, None].repeat(4, 1))

    ws = [
        tapmat(p['w1']), p['b1'].reshape(1, -1),
        tapmat(p['w2']), p['b2'].reshape(1, -1),
        tapmat(p['w3']), p['b3'].reshape(1, -1),
        tapmat(p['ra0']), p['rba0'].reshape(1, -1),
        tapmat(p['rb0']), p['rbb0'].reshape(1, -1),
        tapmat(p['ra1']), p['rba1'].reshape(1, -1),
        tapmat(p['rb1']), p['rbb1'].reshape(1, -1),
        wps, bpf,
    ]
    nblk = N_TOTAL // EBLK
    w_specs = [pl.BlockSpec(w.shape, lambda i, nd=w.ndim: (0,) * nd) for w in ws]
    return pl.pallas_call(
        _enc_kernel,
        grid=(nblk,),
        in_specs=[pl.BlockSpec((EBLK, PATCH), lambda i: (i, 0))] + w_specs,
        out_specs=pl.BlockSpec((EBLK, CODE_DIM), lambda i: (i, 0)),
        out_shape=jax.ShapeDtypeStruct((N_TOTAL, CODE_DIM), jnp.float32),
    )(xpatch, *ws)


def _vq_kernel(z_ref, a_ref, cb_ref, c2_ref, iif_ref, idx_ref, ls_ref):
    z = z_ref[...]
    a = a_ref[...]
    cb = cb_ref[...]
    c2 = c2_ref[...]
    iif = iif_ref[...]
    zb = z * (-2.0)
    b2 = jax.lax.dot_general(zb, cb, dimension_numbers=(((1,), (1,)), ((), ())),
                             preferred_element_type=jnp.float32)
    d = (a + b2) + c2
    m = jnp.min(d, axis=1, keepdims=True)
    idx = jnp.min(jnp.where(d == m, iif, jnp.float32(K)), axis=1)
    idx_ref[0, 0, :] = idx.astype(jnp.int32)
    ls_ref[0, 0, :] = jnp.zeros((8,), jnp.float32) + jnp.sum(m)


def _vq_argmin(z_all, a_all, codebook, c2):
    n = z_all.shape[0]
    nblk = n // RBLK
    c2r = c2.reshape(1, K)
    iif = jnp.arange(K, dtype=jnp.float32).reshape(1, K)
    idx3, ls3 = pl.pallas_call(
        _vq_kernel,
        grid=(nblk,),
        in_specs=[
            pl.BlockSpec((RBLK, CODE_DIM), lambda i: (i, 0)),
            pl.BlockSpec((RBLK, 1), lambda i: (i, 0)),
            pl.BlockSpec((K, CODE_DIM), lambda i: (0, 0)),
            pl.BlockSpec((1, K), lambda i: (0, 0)),
            pl.BlockSpec((1, K), lambda i: (0, 0)),
        ],
        out_specs=[
            pl.BlockSpec((1, 1, RBLK), lambda i: (i, 0, 0)),
            pl.BlockSpec((1, 1, 8), lambda i: (i, 0, 0)),
        ],
        out_shape=[
            jax.ShapeDtypeStruct((nblk, 1, RBLK), jnp.int32),
            jax.ShapeDtypeStruct((nblk, 1, 8), jnp.float32),
        ],
    )(z_all, a_all, codebook, c2r, iif)
    return idx3.reshape(n), jnp.sum(ls3[:, 0, 0])


def kernel(x, params, codebook):
    B, T, C = x.shape
    nump = T // PATCH
    n = B * nump

    xpatch = x[:, :nump * PATCH, :].reshape(B, nump, PATCH, C)
    xpatch = xpatch.transpose(0, 1, 3, 2).reshape(n * C, PATCH)

    z_all = _encode(xpatch, params)

    a_all = jnp.sum(z_all * z_all, axis=1, keepdims=True)
    c2 = jnp.sum(codebook * codebook, axis=1)

    idx_all, sse = _vq_argmin(z_all, a_all, codebook, c2)

    zq = jnp.take(codebook, idx_all, axis=0)
    zq_st = z_all + (zq - z_all)
    z_q = zq_st.reshape(B, nump, C, CODE_DIM)
    indices = idx_all.reshape(B, nump, C)[..., None]
    vq_loss = (1.0 + BETA) * sse / (C * n * CODE_DIM)
    return indices, vq_loss, z_q


# speedup vs baseline: 1.0130x; 1.0130x over previous
import jax, jax.numpy as jnp
from jax.experimental import pallas as pl

PATCH = 16
EMB = 16
CF = 4
K = 8192
H = 64
RH = 32
NRES = 2
BETA = 0.25
CODE_DIM = EMB * (PATCH // CF)

RBLK = 1024
EBLK = 2048
N_TOTAL = 28672


def _bf(v):
    return v.astype(jnp.bfloat16)


def _enc_kernel(x_ref, w1_ref, b1_ref, w2_ref, b2_ref, w3_ref, b3_ref,
                wa0_ref, ba0_ref, wb0_ref, bb0_ref, wa1_ref, ba1_ref,
                wb1_ref, bb1_ref, wp_ref, bp_ref, z_ref):
    xb = _bf(x_ref[...])
    n = xb.shape[0]
    zero1 = jnp.zeros((n, 1), jnp.bfloat16)

    def dot(a, w):
        return jax.lax.dot_general(a, w, dimension_numbers=(((1,), (0,)), ((), ())),
                                   preferred_element_type=jnp.float32)

    w1 = w1_ref[...]
    b1 = b1_ref[...]
    h1 = []
    for lo in range(8):
        if lo == 0:
            win = jnp.concatenate([zero1, xb[:, 0:3]], axis=1)
        elif lo == 7:
            win = jnp.concatenate([xb[:, 13:16], zero1], axis=1)
        else:
            win = xb[:, 2 * lo - 1:2 * lo + 3]
        h1.append(_bf(jax.nn.relu(dot(win, w1) + b1)))

    w2 = w2_ref[...]
    b2 = b2_ref[...]
    zero32 = jnp.zeros((n, 32), jnp.bfloat16)
    h1p = [zero32] + h1 + [zero32]
    h2 = []
    for lo in range(4):
        win = jnp.concatenate(h1p[2 * lo:2 * lo + 4], axis=1)
        h2.append(_bf(jax.nn.relu(dot(win, w2) + b2)))

    w3 = w3_ref[...]
    b3 = b3_ref[...]
    zero64 = jnp.zeros((n, 64), jnp.bfloat16)
    h2p = [zero64] + h2 + [zero64]
    h = [dot(jnp.concatenate(h2p[lo:lo + 3], axis=1), w3) + b3 for lo in range(4)]

    for wa_ref, ba_ref, wb_ref, bb_ref in ((wa0_ref, ba0_ref, wb0_ref, bb0_ref),
                                           (wa1_ref, ba1_ref, wb1_ref, bb1_ref)):
        wa = wa_ref[...]
        ba = ba_ref[...]
        wb = wb_ref[...]
        bb = bb_ref[...]
        rin = [_bf(jax.nn.relu(t)) for t in h]
        rinp = [zero64] + rin + [zero64]
        s = [_bf(jax.nn.relu(dot(jnp.concatenate(rinp[lo:lo + 3], axis=1), wa) + ba))
             for lo in range(4)]
        r = [dot(t, wb) + bb for t in s]
        h = [u + v for u, v in zip(h, r)]

    bpf = bp_ref[...]
    wp = wp_ref[...]
    r = [_bf(jax.nn.relu(t)) for t in h]
    acc = dot(r[0], wp[0])
    for pos in range(1, 4):
        acc = acc + dot(r[pos], wp[pos])
    z_ref[...] = acc + bpf


def _encode(xpatch, params):
    p = params

    def tapmat(w):
        return _bf(w.transpose(2, 1, 0).reshape(-1, w.shape[0]))

    wpt = tapmat(p['wp'])
    wps = jnp.zeros((4, H, CODE_DIM), jnp.bfloat16)
    wps = wps.at[jnp.arange(4)[:, None], :, jnp.arange(EMB)[None, :] * CF
                 + jnp.arange(4)[:, None]].set(wpt.T[None, :, :].repeat(4, 0))
    bpf = jnp.zeros((1, CODE_DIM), jnp.float32)
    bpf = bpf.at[0, jnp.arange(EMB)[:, None] * CF + jnp.arange(4)[None, :]].set(
        p['bp'][:, None].repeat(4, 1))

    ws = [
        tapmat(p['w1']), p['b1'].reshape(1, -1),
        tapmat(p['w2']), p['b2'].reshape(1, -1),
        tapmat(p['w3']), p['b3'].reshape(1, -1),
        tapmat(p['ra0']), p['rba0'].reshape(1, -1),
        tapmat(p['rb0']), p['rbb0'].reshape(1, -1),
        tapmat(p['ra1']), p['rba1'].reshape(1, -1),
        tapmat(p['rb1']), p['rbb1'].reshape(1, -1),
        wps, bpf,
    ]
    nblk = N_TOTAL // EBLK
    w_specs = [pl.BlockSpec(w.shape, lambda i, nd=w.ndim: (0,) * nd) for w in ws]
    return pl.pallas_call(
        _enc_kernel,
        grid=(nblk,),
        in_specs=[pl.BlockSpec((EBLK, PATCH), lambda i: (i, 0))] + w_specs,
        out_specs=pl.BlockSpec((EBLK, CODE_DIM), lambda i: (i, 0)),
        out_shape=jax.ShapeDtypeStruct((N_TOTAL, CODE_DIM), jnp.float32),
    )(xpatch, *ws)


def _vq_kernel(z_ref, a_ref, cb_ref, c2_ref, iif_ref, idx_ref, ls_ref):
    z = z_ref[...]
    a = a_ref[...]
    cb = cb_ref[...]
    c2 = c2_ref[...]
    iif = iif_ref[...]
    zb = z * (-2.0)
    b2 = jax.lax.dot_general(zb, cb, dimension_numbers=(((1,), (1,)), ((), ())),
                             preferred_element_type=jnp.float32)
    d = (a + b2) + c2
    m = jnp.min(d, axis=1, keepdims=True)
    idx = jnp.min(jnp.where(d == m, iif, jnp.float32(K)), axis=1)
    idx_ref[0, 0, :] = idx.astype(jnp.int32)
    ls_ref[0, 0, :] = jnp.zeros((8,), jnp.float32) + jnp.sum(m)


def _vq_argmin(z_all, a_all, codebook, c2):
    n = z_all.shape[0]
    nblk = n // RBLK
    c2r = c2.reshape(1, K)
    iif = jnp.arange(K, dtype=jnp.float32).reshape(1, K)
    idx3, ls3 = pl.pallas_call(
        _vq_kernel,
        grid=(nblk,),
        in_specs=[
            pl.BlockSpec((RBLK, CODE_DIM), lambda i: (i, 0)),
            pl.BlockSpec((RBLK, 1), lambda i: (i, 0)),
            pl.BlockSpec((K, CODE_DIM), lambda i: (0, 0)),
            pl.BlockSpec((1, K), lambda i: (0, 0)),
            pl.BlockSpec((1, K), lambda i: (0, 0)),
        ],
        out_specs=[
            pl.BlockSpec((1, 1, RBLK), lambda i: (i, 0, 0)),
            pl.BlockSpec((1, 1, 8), lambda i: (i, 0, 0)),
        ],
        out_shape=[
            jax.ShapeDtypeStruct((nblk, 1, RBLK), jnp.int32),
            jax.ShapeDtypeStruct((nblk, 1, 8), jnp.float32),
        ],
    )(z_all, a_all, codebook, c2r, iif)
    return idx3.reshape(n), jnp.sum(ls3[:, 0, 0])


def kernel(x, params, codebook):
    B, T, C = x.shape
    nump = T // PATCH
    n = B * nump

    xpatch = x[:, :nump * PATCH, :].reshape(B, nump, PATCH, C)
    xpatch = xpatch.transpose(0, 1, 3, 2).reshape(n * C, PATCH)

    z_all = _encode(xpatch, params)

    a_all = jnp.sum(z_all * z_all, axis=1, keepdims=True)
    c2 = jnp.sum(codebook * codebook, axis=1)

    idx_all, sse = _vq_argmin(z_all, a_all, codebook, c2)

    zq = jnp.take(codebook, idx_all, axis=0)
    zq_st = z_all + (zq - z_all)
    z_q = zq_st.reshape(B, nump, C, CODE_DIM)
    indices = idx_all.reshape(B, nump, C)[..., None]
    vq_loss = (1.0 + BETA) * sse / (C * n * CODE_DIM)
    return indices, vq_loss, z_q
